# baseline (speedup 1.0000x reference)
import functools

import jax
import jax.numpy as jnp
from jax import lax
from jax.experimental import pallas as pl
from jax.experimental.pallas import tpu as pltpu
from jax.experimental.pallas import tpu_sc as plsc

N = 10000
D = 128
H = 128
C = 64
E = 320000

NC = 2
NS = 16
NW = NC * NS
EPW = E // NW
K = 80
NCHUNK = EPW // K
DAUG = 144
ROWS_PER_TILE = N // NS


def _mk_table(h):
    nr = jnp.sqrt(jnp.sum(h * h, axis=1, keepdims=True)) + 1e-12
    pad = jnp.zeros((h.shape[0], DAUG - H - 1), jnp.float32)
    return jnp.concatenate([h / nr, nr, pad], axis=1)


def _proj_body(x_ref, w_ref, b_ref, tab_ref):
    h = jnp.dot(x_ref[...], w_ref[...], preferred_element_type=jnp.float32)
    h = jnp.maximum(h + b_ref[...], 0.0)
    tab_ref[...] = _mk_table(h)


def _proj(x, w1, b1):
    return pl.pallas_call(
        _proj_body,
        out_shape=jax.ShapeDtypeStruct((N, DAUG), jnp.float32),
    )(x, w1, b1)


def _combine_body(u_ref, tab_ref):
    s = u_ref[0] + u_ref[1]
    h = s[:, :H] / (s[:, H:H + 1] + 1e-12)
    tab_ref[...] = _mk_table(h)


def _combine(u):
    return pl.pallas_call(
        _combine_body,
        out_shape=jax.ShapeDtypeStruct((N, DAUG), jnp.float32),
    )(u)


def _shuffle_bf16(tab):
    hn = tab[:, :H]
    y = jnp.transpose(hn.reshape(N, H // 32, 2, 16), (0, 1, 3, 2))
    return y.reshape(N, H).astype(jnp.bfloat16)


def _final_body(tab_ref, w_ref, b_ref, out_ref):
    h = tab_ref[:, :H] * tab_ref[:, H:H + 1]
    out_ref[...] = (
        jnp.dot(h, w_ref[...], preferred_element_type=jnp.float32) + b_ref[...]
    )


def _final(tab, w2, b2):
    return pl.pallas_call(
        _final_body,
        out_shape=jax.ShapeDtypeStruct((N, C), jnp.float32),
    )(tab, w2, b2)


_MESH = plsc.VectorSubcoreMesh(core_axis_name="c", subcore_axis_name="s")


@functools.partial(
    pl.kernel,
    mesh=_MESH,
    out_type=jax.ShapeDtypeStruct((NC, N, DAUG), jnp.float32),
    scratch_types=[
        pltpu.VMEM((16,), jnp.float32),
        pltpu.VMEM((K,), jnp.int32),
        pltpu.VMEM((K,), jnp.int32),
        pltpu.VMEM((K,), jnp.int32),
        pltpu.VMEM((K,), jnp.int32),
        pltpu.VMEM((K, DAUG), jnp.float32),
        pltpu.VMEM((K, DAUG), jnp.float32),
        pltpu.VMEM((K, DAUG), jnp.float32),
        pltpu.VMEM((K, H), jnp.bfloat16),
        pltpu.VMEM((K,), jnp.int32),
        pltpu.VMEM((K,), jnp.int32),
        pltpu.VMEM_SHARED((N, DAUG), jnp.float32),
        pltpu.SemaphoreType.DMA,
        pltpu.SemaphoreType.DMA,
        pltpu.SemaphoreType.DMA,
        pltpu.SemaphoreType.DMA,
        pltpu.SemaphoreType.DMA,
        pltpu.SemaphoreType.DMA,
        pltpu.SemaphoreType.DMA,
        pltpu.SemaphoreType.DMA,
    ],
    compiler_params=pltpu.CompilerParams(
        use_tc_tiling_on_sc=False, needs_layout_passes=False),
)
def _edge_kernel(tab_hbm, hnb_hbm, src_hbm, dst_hbm, par_hbm, out_hbm,
                 par, scs0, scs1, scat0, scat1, rs0, rs1, zb, rd,
                 scatT0, scatT1, u_sh, sem_s0, sem_s1, sem_d,
                 sem_i0, sem_i1, sem_sc, sem_t0, sem_t1):
    cid = lax.axis_index("c")
    sid = lax.axis_index("s")
    wid = cid * NS + sid
    base = pl.multiple_of(wid * EPW, 8)

    pltpu.sync_copy(par_hbm, par)

    zero16 = jnp.zeros((16,), jnp.float32)
    lanes = lax.iota(jnp.int32, 16)

    def _zr(r, c):
        for j in range(DAUG // 16):
            zb[r, pl.ds(j * 16, 16)] = zero16
        return c
    lax.fori_loop(0, K, _zr, 0)
    rbase = sid * ROWS_PER_TILE
    for c7 in range(ROWS_PER_TILE // K):
        pltpu.sync_copy(zb, u_sh.at[pl.ds(rbase + c7 * K, K)])
    _rem = ROWS_PER_TILE % K
    if _rem:
        pltpu.sync_copy(zb.at[pl.ds(0, _rem)],
                        u_sh.at[pl.ds(rbase + (ROWS_PER_TILE // K) * K, _rem)])

    plsc.subcore_barrier()

    pvec = par[pl.ds(0, 16)]
    beta = pvec[0]
    shift = pvec[1]

    def _edges(rs, lo, hi):
        def _pg(t, c):
            o16 = pl.multiple_of(t * 16, 8)
            for i in range(16):
                e = o16 + i
                s = [rs[e, pl.ds(jj * 16, 16)] for jj in range(H // 16)]
                nr16 = rs[e, pl.ds(H, 16)]
                ts = []
                for b4 in range(H // 32):
                    pa, pb = plsc.unpack(
                        rd[e, pl.ds(b4 * 32, 32)],
                        format=plsc.PackFormat.INTERLEAVED,
                        preferred_element_type=jnp.float32)
                    ts.append(s[2 * b4] * pa + s[2 * b4 + 1] * pb)
                cos = jnp.sum((ts[0] + ts[1]) + (ts[2] + ts[3]))
                exv = jnp.exp(jnp.full((16,), beta * cos - shift, jnp.float32))
                wv = exv * jnp.full((16,), nr16[0], jnp.float32)
                for jj in range(H // 16):
                    rs[e, pl.ds(jj * 16, 16)] = s[jj] * wv
                rs[e, pl.ds(H, 16)] = jnp.where(lanes == 0, exv, zero16)
            return c
        lax.fori_loop(lo, hi, _pg, 0)


    pltpu.sync_copy(src_hbm.at[pl.ds(base, K)], scs0)
    pltpu.sync_copy(dst_hbm.at[pl.ds(base, K)], scat0)
    pltpu.async_copy(dst_hbm.at[pl.ds(base, K)], scatT0, sem_t0)
    pltpu.async_copy(tab_hbm.at[scs0], rs0, sem_s0)
    pltpu.async_copy(hnb_hbm.at[scat0], rd, sem_d)
    off1p = pl.multiple_of(base + K, 8)
    pltpu.async_copy(src_hbm.at[pl.ds(off1p, K)], scs1, sem_i1)
    pltpu.async_copy(dst_hbm.at[pl.ds(off1p, K)], scat1, sem_i1)

    def _section(j, first, scs_c, scat_c, scatT_c, rs_c, sem_c, sem_ic,
                 sem_tc, scs_n, scat_n, scatT_n, rs_n, sem_n, sem_in,
                 sem_tn):
        pltpu.make_async_copy(hnb_hbm.at[scat_c], rd, sem_d).wait()
        pltpu.make_async_copy(tab_hbm.at[scs_c], rs_c, sem_c).wait()
        off1 = pl.multiple_of(base + (j + 1) * K, 8)
        pltpu.make_async_copy(src_hbm.at[pl.ds(off1, K)], scs_n, sem_in).wait()
        pltpu.make_async_copy(dst_hbm.at[pl.ds(off1, K)], scat_n, sem_in).wait()
        _edges(rs_c, 0, 2)
        if first:
            @pl.when(j > 0)
            def _():
                pltpu.make_async_copy(rs_n, u_sh.at[scatT_n], sem_sc).wait()
        else:
            pltpu.make_async_copy(rs_n, u_sh.at[scatT_n], sem_sc).wait()
        pltpu.async_copy(tab_hbm.at[scs_n], rs_n, sem_n)
        pltpu.async_copy(hnb_hbm.at[scat_n], rd, sem_d)
        off2 = pl.multiple_of(
            jnp.where(j + 2 < NCHUNK, base + (j + 2) * K, base), 8)
        pltpu.async_copy(src_hbm.at[pl.ds(off2, K)], scs_c, sem_ic)
        pltpu.async_copy(dst_hbm.at[pl.ds(off2, K)], scat_c, sem_ic)
        pltpu.async_copy(dst_hbm.at[pl.ds(off1, K)], scatT_n, sem_tn)
        _edges(rs_c, 2, K // 16)
        pltpu.make_async_copy(dst_hbm.at[pl.ds(off1, K)], scatT_c, sem_tc).wait()
        pltpu.async_copy(rs_c, u_sh.at[scatT_c], sem_sc, add=True)

    def _pair(m, carry):
        _section(2 * m, True, scs0, scat0, scatT0, rs0, sem_s0, sem_i0,
                 sem_t0, scs1, scat1, scatT1, rs1, sem_s1, sem_i1, sem_t1)
        _section(2 * m + 1, False, scs1, scat1, scatT1, rs1, sem_s1, sem_i1,
                 sem_t1, scs0, scat0, scatT0, rs0, sem_s0, sem_i0, sem_t0)
        return carry

    lax.fori_loop(0, NCHUNK // 2, _pair, 0)

    pltpu.make_async_copy(hnb_hbm.at[scat0], rd, sem_d).wait()
    pltpu.make_async_copy(tab_hbm.at[scs0], rs0, sem_s0).wait()
    pltpu.make_async_copy(src_hbm.at[pl.ds(base, K)], scs1, sem_i1).wait()
    pltpu.make_async_copy(dst_hbm.at[pl.ds(base, K)], scat1, sem_i1).wait()
    pltpu.make_async_copy(rs1, u_sh.at[scatT1], sem_sc).wait()
    _edges(rs0, 0, K // 16)
    pltpu.make_async_copy(dst_hbm.at[pl.ds(base, K)], scatT0, sem_t0).wait()
    pltpu.sync_copy(rs0, u_sh.at[scatT0], add=True)

    plsc.subcore_barrier()

    pltpu.sync_copy(u_sh.at[pl.ds(rbase, ROWS_PER_TILE)],
                    out_hbm.at[cid, pl.ds(rbase, ROWS_PER_TILE)])


def kernel(features, edge_index, W1, b1, betas, W2, b2):
    src = edge_index[0]
    dst = edge_index[1]
    tab = _proj(features, W1, b1)

    def _layer(tab, beta):
        par = (jnp.zeros((16,), jnp.float32)
               .at[0].set(beta).at[1].set(jnp.abs(beta)))
        u = _edge_kernel(tab, _shuffle_bf16(tab), src, dst, par)
        return _combine(u), 0

    tab, _ = lax.scan(_layer, tab, betas)
    return _final(tab, W2, b2)


# speedup vs baseline: 2.1280x; 2.1280x over previous
import functools

import jax
import jax.numpy as jnp
from jax import lax
from jax.experimental import pallas as pl
from jax.experimental.pallas import tpu as pltpu
from jax.experimental.pallas import tpu_sc as plsc

N = 10000
D = 128
H = 128
C = 64
E = 320000

NC = 2
NS = 16
NW = NC * NS
EPW = E // NW
K = 80
NCHUNK = EPW // K
DAUG = 144
ROWS_PER_TILE = N // NS


def _mk_table(h):
    nr = jnp.sqrt(jnp.sum(h * h, axis=1, keepdims=True)) + 1e-12
    pad = jnp.zeros((h.shape[0], DAUG - H - 1), jnp.float32)
    return jnp.concatenate([h / nr, nr, pad], axis=1)


def _proj_body(x_ref, w_ref, b_ref, tab_ref):
    h = jnp.dot(x_ref[...], w_ref[...], preferred_element_type=jnp.float32)
    h = jnp.maximum(h + b_ref[...], 0.0)
    tab_ref[...] = _mk_table(h)


def _proj(x, w1, b1):
    return pl.pallas_call(
        _proj_body,
        out_shape=jax.ShapeDtypeStruct((N, DAUG), jnp.float32),
    )(x, w1, b1)


def _combine_body(u_ref, tab_ref):
    s = u_ref[0] + u_ref[1]
    h = s[:, :H] / (s[:, H:H + 1] + 1e-12)
    tab_ref[...] = _mk_table(h)


def _combine(u):
    return pl.pallas_call(
        _combine_body,
        out_shape=jax.ShapeDtypeStruct((N, DAUG), jnp.float32),
    )(u)


def _shuffle_bf16(tab):
    hn = tab[:, :H]
    y = jnp.transpose(hn.reshape(N, H // 32, 2, 16), (0, 1, 3, 2))
    return y.reshape(N, H).astype(jnp.bfloat16)


def _final_body(tab_ref, w_ref, b_ref, out_ref):
    h = tab_ref[:, :H] * tab_ref[:, H:H + 1]
    out_ref[...] = (
        jnp.dot(h, w_ref[...], preferred_element_type=jnp.float32) + b_ref[...]
    )


def _final(tab, w2, b2):
    return pl.pallas_call(
        _final_body,
        out_shape=jax.ShapeDtypeStruct((N, C), jnp.float32),
    )(tab, w2, b2)


_MESH = plsc.VectorSubcoreMesh(core_axis_name="c", subcore_axis_name="s")


@functools.partial(
    pl.kernel,
    mesh=_MESH,
    out_type=jax.ShapeDtypeStruct((NC, N, DAUG), jnp.float32),
    scratch_types=[
        pltpu.VMEM((16,), jnp.float32),
        pltpu.VMEM((K,), jnp.int32),
        pltpu.VMEM((K,), jnp.int32),
        pltpu.VMEM((K,), jnp.int32),
        pltpu.VMEM((K,), jnp.int32),
        pltpu.VMEM((K, DAUG), jnp.float32),
        pltpu.VMEM((K, DAUG), jnp.float32),
        pltpu.VMEM((K, H), jnp.bfloat16),
        pltpu.VMEM((K, H), jnp.bfloat16),
        pltpu.VMEM((K,), jnp.float32),
        pltpu.VMEM((K,), jnp.float32),
        pltpu.VMEM((K,), jnp.int32),
        pltpu.VMEM((K,), jnp.int32),
        pltpu.VMEM_SHARED((N, DAUG), jnp.float32),
        pltpu.SemaphoreType.DMA,
        pltpu.SemaphoreType.DMA,
        pltpu.SemaphoreType.DMA,
        pltpu.SemaphoreType.DMA,
        pltpu.SemaphoreType.DMA,
        pltpu.SemaphoreType.DMA,
        pltpu.SemaphoreType.DMA,
        pltpu.SemaphoreType.DMA,
        pltpu.SemaphoreType.DMA,
    ],
    compiler_params=pltpu.CompilerParams(
        use_tc_tiling_on_sc=False, needs_layout_passes=False),
)
def _edge_kernel(tab_hbm, hnb_hbm, src_hbm, dst_hbm, par_hbm, out_hbm,
                 par, scs0, scs1, scat0, scat1, rs0, rs1, rd0, rd1,
                 exb, wb, scatT0, scatT1, u_sh, sem_s0, sem_s1, sem_d0,
                 sem_d1, sem_i0, sem_i1, sem_sc, sem_t0, sem_t1):
    cid = lax.axis_index("c")
    sid = lax.axis_index("s")
    wid = cid * NS + sid
    base = pl.multiple_of(wid * EPW, 8)

    pltpu.sync_copy(par_hbm, par)

    zero16 = jnp.zeros((16,), jnp.float32)
    lanes = lax.iota(jnp.int32, 16)

    def _zr(r, c):
        for j in range(DAUG // 16):
            rs0[r, pl.ds(j * 16, 16)] = zero16
        return c
    lax.fori_loop(0, K, _zr, 0)
    rbase = sid * ROWS_PER_TILE
    for c7 in range(ROWS_PER_TILE // K):
        pltpu.sync_copy(rs0, u_sh.at[pl.ds(rbase + c7 * K, K)])
    _rem = ROWS_PER_TILE % K
    if _rem:
        pltpu.sync_copy(rs0.at[pl.ds(0, _rem)],
                        u_sh.at[pl.ds(rbase + (ROWS_PER_TILE // K) * K, _rem)])

    plsc.subcore_barrier()

    pvec = par[pl.ds(0, 16)]
    beta = pvec[0]
    shift = pvec[1]

    def _phase_a(rs, rd):
        def _pa(t, c):
            o16 = pl.multiple_of(t * 16, 8)
            cosv = zero16
            nrv = zero16
            for i in range(16):
                e = o16 + i
                acc = zero16
                for b4 in range(H // 32):
                    pa, pb = plsc.unpack(
                        rd[e, pl.ds(b4 * 32, 32)],
                        format=plsc.PackFormat.INTERLEAVED,
                        preferred_element_type=jnp.float32)
                    acc = (acc + rs[e, pl.ds(b4 * 32, 16)] * pa
                           + rs[e, pl.ds(b4 * 32 + 16, 16)] * pb)
                cosv = jnp.where(lanes == i, jnp.sum(acc), cosv)
                nrv = jnp.where(lanes == i, rs[e, pl.ds(H, 16)][0], nrv)
            ex = jnp.exp(beta * cosv - shift)
            exb[pl.ds(o16, 16)] = ex
            wb[pl.ds(o16, 16)] = ex * nrv
            return c
        lax.fori_loop(0, K // 16, _pa, 0)

    def _phase_b(rs):
        def _pb(t, c):
            o16 = pl.multiple_of(t * 16, 8)
            wvec = wb[pl.ds(o16, 16)]
            exvec = exb[pl.ds(o16, 16)]
            for i in range(16):
                e = o16 + i
                wv = jnp.full((16,), wvec[i], jnp.float32)
                for jj in range(H // 16):
                    rs[e, pl.ds(jj * 16, 16)] = rs[e, pl.ds(jj * 16, 16)] * wv
                rs[e, pl.ds(H, 16)] = jnp.where(
                    lanes == 0, jnp.full((16,), exvec[i], jnp.float32), zero16)
            return c
        lax.fori_loop(0, K // 16, _pb, 0)


    pltpu.sync_copy(src_hbm.at[pl.ds(base, K)], scs0)
    pltpu.sync_copy(dst_hbm.at[pl.ds(base, K)], scat0)
    pltpu.async_copy(dst_hbm.at[pl.ds(base, K)], scatT0, sem_t0)
    pltpu.async_copy(tab_hbm.at[scs0], rs0, sem_s0)
    pltpu.async_copy(hnb_hbm.at[scat0], rd0, sem_d0)
    off1p = pl.multiple_of(base + K, 8)
    pltpu.async_copy(src_hbm.at[pl.ds(off1p, K)], scs1, sem_i1)
    pltpu.async_copy(dst_hbm.at[pl.ds(off1p, K)], scat1, sem_i1)

    def _section(j, first, scs_c, scat_c, scatT_c, rs_c, rd_c, sem_c,
                 sem_dc, sem_ic, sem_tc, scs_n, scat_n, scatT_n, rs_n,
                 rd_n, sem_n, sem_dn, sem_in, sem_tn):
        pltpu.make_async_copy(hnb_hbm.at[scat_c], rd_c, sem_dc).wait()
        pltpu.make_async_copy(tab_hbm.at[scs_c], rs_c, sem_c).wait()
        off1 = pl.multiple_of(base + (j + 1) * K, 8)
        pltpu.make_async_copy(src_hbm.at[pl.ds(off1, K)], scs_n, sem_in).wait()
        pltpu.make_async_copy(dst_hbm.at[pl.ds(off1, K)], scat_n, sem_in).wait()
        if first:
            @pl.when(j > 0)
            def _():
                pltpu.make_async_copy(rs_n, u_sh.at[scatT_n], sem_sc).wait()
        else:
            pltpu.make_async_copy(rs_n, u_sh.at[scatT_n], sem_sc).wait()
        pltpu.async_copy(tab_hbm.at[scs_n], rs_n, sem_n)
        pltpu.async_copy(hnb_hbm.at[scat_n], rd_n, sem_dn)
        off2 = pl.multiple_of(
            jnp.where(j + 2 < NCHUNK, base + (j + 2) * K, base), 8)
        pltpu.async_copy(src_hbm.at[pl.ds(off2, K)], scs_c, sem_ic)
        pltpu.async_copy(dst_hbm.at[pl.ds(off2, K)], scat_c, sem_ic)
        pltpu.async_copy(dst_hbm.at[pl.ds(off1, K)], scatT_n, sem_tn)
        _phase_a(rs_c, rd_c)
        _phase_b(rs_c)
        pltpu.make_async_copy(dst_hbm.at[pl.ds(off1, K)], scatT_c, sem_tc).wait()
        pltpu.async_copy(rs_c, u_sh.at[scatT_c], sem_sc, add=True)

    def _pair(m, carry):
        _section(2 * m, True, scs0, scat0, scatT0, rs0, rd0, sem_s0,
                 sem_d0, sem_i0, sem_t0, scs1, scat1, scatT1, rs1, rd1,
                 sem_s1, sem_d1, sem_i1, sem_t1)
        _section(2 * m + 1, False, scs1, scat1, scatT1, rs1, rd1, sem_s1,
                 sem_d1, sem_i1, sem_t1, scs0, scat0, scatT0, rs0, rd0,
                 sem_s0, sem_d0, sem_i0, sem_t0)
        return carry

    lax.fori_loop(0, NCHUNK // 2, _pair, 0)

    pltpu.make_async_copy(hnb_hbm.at[scat0], rd0, sem_d0).wait()
    pltpu.make_async_copy(tab_hbm.at[scs0], rs0, sem_s0).wait()
    pltpu.make_async_copy(src_hbm.at[pl.ds(base, K)], scs1, sem_i1).wait()
    pltpu.make_async_copy(dst_hbm.at[pl.ds(base, K)], scat1, sem_i1).wait()
    pltpu.make_async_copy(rs1, u_sh.at[scatT1], sem_sc).wait()
    _phase_a(rs0, rd0)
    _phase_b(rs0)
    pltpu.make_async_copy(dst_hbm.at[pl.ds(base, K)], scatT0, sem_t0).wait()
    pltpu.sync_copy(rs0, u_sh.at[scatT0], add=True)

    plsc.subcore_barrier()

    pltpu.sync_copy(u_sh.at[pl.ds(rbase, ROWS_PER_TILE)],
                    out_hbm.at[cid, pl.ds(rbase, ROWS_PER_TILE)])


def kernel(features, edge_index, W1, b1, betas, W2, b2):
    src = edge_index[0]
    dst = edge_index[1]
    tab = _proj(features, W1, b1)

    def _layer(tab, beta):
        par = (jnp.zeros((16,), jnp.float32)
               .at[0].set(beta).at[1].set(jnp.abs(beta)))
        u = _edge_kernel(tab, _shuffle_bf16(tab), src, dst, par)
        return _combine(u), 0

    tab, _ = lax.scan(_layer, tab, betas)
    return _final(tab, W2, b2)
